# Initial kernel scaffold; baseline (speedup 1.0000x reference)
import jax
import jax.numpy as jnp
from jax.experimental import pallas as pl


def kernel(id, edge_index, x, index, info_index, text_index, image_index, text_fc_W, text_fc_b, image_fc_W, image_fc_b, feat_fc_W, feat_fc_b, dir_emb, weight, gcn_W1, gcn_b1, gcn_W2, gcn_b2, gcn_W3, gcn_b3, cls_W, cls_b, reg_W, reg_b):
    raise NotImplementedError("write your pallas kernel here")


# speedup vs baseline: 16.7820x; 16.7820x over previous
import functools

import jax
import jax.numpy as jnp
from jax import lax
from jax.experimental import pallas as pl
from jax.experimental.pallas import tpu as pltpu
from jax.experimental.pallas import tpu_sc as plsc

_N = 50000
_E = 800000
_IN = 256
_D = 64
_B = 16384
_NDIR = 120
_OUT = 6

_NC, _NS = 2, 16
_NW = _NC * _NS
_NQ, _FQ = 4, 16
_SPR = 51200
_EP = 802816
_CH = 128
_GRP = 14

_EW = _EP // _NW
_ET = _EP // _NS
_BW = _B // _NW
_BS = _B // _NS

_LAST = (_NS - 1) * 3128


def _mk_mesh():
    return plsc.VectorSubcoreMesh(core_axis_name="c", subcore_axis_name="s",
                                  num_cores=_NC, num_subcores=_NS)


_SC_PARAMS = pltpu.CompilerParams(use_tc_tiling_on_sc=False)


def _fill_rows16(ref, rows, value):
    v = jnp.full((16,), value, jnp.float32)

    @pl.loop(0, rows)
    def _(r):
        ref[r, pl.ds(0, 16)] = v


def _ka_body(dst_hbm, info_hbm, text_hbm, title_hbm, x_hbm,
             deg_out, xinfo_out, xtext_out, xtitle_out,
             *rest):
    didx = rest[0:_GRP]
    gidx = rest[_GRP]
    ones_v = rest[_GRP + 1]
    zbuf = rest[_GRP + 2]
    rows = rest[_GRP + 3]
    deg_sh = rest[_GRP + 4]
    lsem, gsem, ssem = rest[_GRP + 5:_GRP + 8]

    c = lax.axis_index("c")
    s = lax.axis_index("s")
    wid = s * _NC + c

    _fill_rows16(zbuf, _CH, 0.0)
    _fill_rows16(ones_v, _CH, 1.0)
    zcp = [pltpu.async_copy(zbuf, deg_sh.at[pl.ds(s * 3200 + k * _CH, _CH)],
                            lsem) for k in range(25)]
    for cp in zcp:
        cp.wait()
    plsc.subcore_barrier()

    ebase = wid * _EW

    @pl.loop(0, _EW // (_GRP * _CH))
    def _(g):
        goff = ebase + g * (_GRP * _CH)
        loads = [pltpu.async_copy(dst_hbm.at[pl.ds(goff + j * _CH, _CH)],
                                  didx[j], lsem) for j in range(_GRP)]
        for cp in loads:
            cp.wait()
        adds = [pltpu.async_copy(ones_v, deg_sh.at[didx[j]], ssem, add=True)
                for j in range(_GRP)]
        for cp in adds:
            cp.wait()

    plsc.subcore_barrier()
    pltpu.sync_copy(deg_sh.at[pl.ds(s * 3200, 3200)],
                    deg_out.at[pl.ds(c * _SPR + s * 3200, 3200)])

    rbase = wid * _BW
    for idx_hbm, out_hbm in ((info_hbm, xinfo_out), (text_hbm, xtext_out),
                             (title_hbm, xtitle_out)):
        @pl.loop(0, _BW // _CH)
        def _(t):
            off = rbase + t * _CH
            pltpu.sync_copy(idx_hbm.at[pl.ds(off, _CH)], gidx)
            pltpu.async_copy(x_hbm.at[gidx], rows, gsem).wait()
            pltpu.sync_copy(rows, out_hbm.at[pl.ds(off, _CH)])


def _sc_prep(dst3, info_index, text_index, title_index, x):
    kern = pl.kernel(
        _ka_body,
        out_type=[
            jax.ShapeDtypeStruct((_NC * _SPR, _FQ), jnp.float32),
            jax.ShapeDtypeStruct((_B, _IN), jnp.float32),
            jax.ShapeDtypeStruct((_B, _IN), jnp.float32),
            jax.ShapeDtypeStruct((_B, _IN), jnp.float32),
        ],
        mesh=_mk_mesh(),
        compiler_params=_SC_PARAMS,
        scratch_types=(
            [pltpu.VMEM((_CH,), jnp.int32) for _ in range(_GRP)]
            + [pltpu.VMEM((_CH,), jnp.int32),
               pltpu.VMEM((_CH, _FQ), jnp.float32),
               pltpu.VMEM((_CH, _FQ), jnp.float32),
               pltpu.VMEM((_CH, _IN), jnp.float32),
               pltpu.VMEM_SHARED((_SPR, _FQ), jnp.float32),
               pltpu.SemaphoreType.DMA,
               pltpu.SemaphoreType.DMA,
               pltpu.SemaphoreType.DMA]
        ),
    )
    return kern(dst3, info_index, text_index, title_index, x)


def _kc_body(src4_hbm, dst_hbm, hw_hbm, seg_out, *rest):
    sidx = rest[0:_GRP]
    didx = rest[_GRP:2 * _GRP]
    rows = rest[2 * _GRP]
    zbuf = rest[2 * _GRP + 1]
    acc = rest[2 * _GRP + 2]
    lsem, gsem, ssem = rest[2 * _GRP + 3:2 * _GRP + 6]

    c = lax.axis_index("c")
    s = lax.axis_index("s")
    ebase = s * _ET

    _fill_rows16(zbuf, _CH, 0.0)

    for qp in range(2):
        q = c * 2 + qp

        zcp = [pltpu.async_copy(zbuf, acc.at[pl.ds(s * 3200 + k * _CH, _CH)],
                                lsem) for k in range(25)]
        for cp in zcp:
            cp.wait()
        plsc.subcore_barrier()

        @pl.loop(0, _ET // (_GRP * _CH))
        def _(g):
            goff = goff0 = g * (_GRP * _CH)
            loads = [pltpu.async_copy(
                src4_hbm.at[pl.ds(q * _EP + ebase + goff + j * _CH, _CH)],
                sidx[j], lsem) for j in range(_GRP)]
            loads += [pltpu.async_copy(
                dst_hbm.at[pl.ds(ebase + goff + j * _CH, _CH)],
                didx[j], lsem) for j in range(_GRP)]
            for cp in loads:
                cp.wait()
            gath = [pltpu.async_copy(hw_hbm.at[sidx[j]], rows.at[j], gsem)
                    for j in range(_GRP)]
            for cp in gath:
                cp.wait()
            adds = [pltpu.async_copy(rows.at[j], acc.at[didx[j]], ssem,
                                     add=True) for j in range(_GRP)]
            for cp in adds:
                cp.wait()

        plsc.subcore_barrier()

        @pl.when(s < _NS - 1)
        def _():
            pltpu.sync_copy(acc.at[pl.ds(s * 3128, 3128)],
                            seg_out.at[pl.ds(q * _N + s * 3128, 3128)])

        @pl.when(s == _NS - 1)
        def _():
            pltpu.sync_copy(acc.at[pl.ds(_LAST, _N - _LAST)],
                            seg_out.at[pl.ds(q * _N + _LAST, _N - _LAST)])

        plsc.subcore_barrier()


def _sc_segsum(src4, dst3, hw4):
    kern = pl.kernel(
        _kc_body,
        out_type=jax.ShapeDtypeStruct((_NQ * _N, _FQ), jnp.float32),
        mesh=_mk_mesh(),
        compiler_params=_SC_PARAMS,
        scratch_types=(
            [pltpu.VMEM((_CH,), jnp.int32) for _ in range(2 * _GRP)]
            + [pltpu.VMEM((_GRP, _CH, _FQ), jnp.float32),
               pltpu.VMEM((_CH, _FQ), jnp.float32),
               pltpu.VMEM_SHARED((_SPR, _FQ), jnp.float32),
               pltpu.SemaphoreType.DMA,
               pltpu.SemaphoreType.DMA,
               pltpu.SemaphoreType.DMA]
        ),
    )
    return kern(src4, dst3, hw4)


def _kf_body(idx4_hbm, idx_hbm, seg_hbm, hw_hbm, dinv_hbm,
             segg_out, hwg_out, dinvg_out, *rest):
    gidx, rows_a, rows_b = rest[0:3]
    gsem = rest[3]

    c = lax.axis_index("c")
    s = lax.axis_index("s")
    rbase = s * _BS

    for qp in range(2):
        q = c * 2 + qp

        @pl.loop(0, _BS // _CH)
        def _(t):
            off = rbase + t * _CH
            pltpu.sync_copy(idx4_hbm.at[pl.ds(q * _B + off, _CH)], gidx)
            pltpu.async_copy(seg_hbm.at[gidx], rows_a, gsem).wait()
            pltpu.sync_copy(rows_a, segg_out.at[pl.ds(q * _B + off, _CH)])
            pltpu.async_copy(hw_hbm.at[gidx], rows_b, gsem).wait()
            pltpu.sync_copy(rows_b, hwg_out.at[pl.ds(q * _B + off, _CH)])

    @pl.when(c == 0)
    def _():
        @pl.loop(0, _BS // _CH)
        def _(t):
            off = rbase + t * _CH
            pltpu.sync_copy(idx_hbm.at[pl.ds(off, _CH)], gidx)
            pltpu.async_copy(dinv_hbm.at[gidx], rows_a, gsem).wait()
            pltpu.sync_copy(rows_a, dinvg_out.at[pl.ds(off, _CH)])


def _sc_final_gather(idx4, index, seg4, hw4, dinv16):
    kern = pl.kernel(
        _kf_body,
        out_type=[
            jax.ShapeDtypeStruct((_NQ * _B, _FQ), jnp.float32),
            jax.ShapeDtypeStruct((_NQ * _B, _FQ), jnp.float32),
            jax.ShapeDtypeStruct((_B, _FQ), jnp.float32),
        ],
        mesh=_mk_mesh(),
        compiler_params=_SC_PARAMS,
        scratch_types=[
            pltpu.VMEM((_CH,), jnp.int32),
            pltpu.VMEM((_CH, _FQ), jnp.float32),
            pltpu.VMEM((_CH, _FQ), jnp.float32),
            pltpu.SemaphoreType.DMA,
        ],
    )
    return kern(idx4, index, seg4, hw4, dinv16)


_RN = 1024
_RB = 1024


def _kb_body(x_ref, deg_ref, fw_ref, fb_ref, w1_ref, hw_ref, dinv_ref):
    dsum = deg_ref[0, :, 0:1] + deg_ref[1, :, 0:1] + 1.0
    dinv = lax.rsqrt(dsum)
    h = jnp.dot(x_ref[...], fw_ref[...],
                preferred_element_type=jnp.float32) + fb_ref[...]
    hw = jnp.dot(h, w1_ref[...], preferred_element_type=jnp.float32) * dinv
    for qq in range(_NQ):
        hw_ref[qq] = hw[:, qq * _FQ:(qq + 1) * _FQ]
    dinv_ref[...] = jnp.broadcast_to(dinv, (_RN, _FQ))


def _tc_first(x, deg2, feat_W, feat_b1, gcn_W1):
    return pl.pallas_call(
        _kb_body,
        grid=(pl.cdiv(_N, _RN),),
        in_specs=[
            pl.BlockSpec((_RN, _IN), lambda i: (i, 0)),
            pl.BlockSpec((_NC, _RN, _FQ), lambda i: (0, i, 0)),
            pl.BlockSpec((_IN, _D), lambda i: (0, 0)),
            pl.BlockSpec((1, _D), lambda i: (0, 0)),
            pl.BlockSpec((_D, _D), lambda i: (0, 0)),
        ],
        out_specs=[
            pl.BlockSpec((_NQ, _RN, _FQ), lambda i: (0, i, 0)),
            pl.BlockSpec((_RN, _FQ), lambda i: (i, 0)),
        ],
        out_shape=[
            jax.ShapeDtypeStruct((_NQ, _N, _FQ), jnp.float32),
            jax.ShapeDtypeStruct((_N, _FQ), jnp.float32),
        ],
    )(x, deg2, feat_W, feat_b1, gcn_W1)


def _kd_body(seg_ref, hw_ref, dinv_ref, b_ref, w_ref, out_ref):
    seg = jnp.concatenate([seg_ref[qq] for qq in range(_NQ)], axis=1)
    hw = jnp.concatenate([hw_ref[qq] for qq in range(_NQ)], axis=1)
    dinv = dinv_ref[:, 0:1]
    hcur = jnp.maximum(dinv * (seg + hw) + b_ref[...], 0.0)
    hwn = jnp.dot(hcur, w_ref[...], preferred_element_type=jnp.float32) * dinv
    for qq in range(_NQ):
        out_ref[qq] = hwn[:, qq * _FQ:(qq + 1) * _FQ]


def _tc_layer(seg4, hwp, dinv16, b_prev1, W_next):
    return pl.pallas_call(
        _kd_body,
        grid=(pl.cdiv(_N, _RN),),
        in_specs=[
            pl.BlockSpec((_NQ, _RN, _FQ), lambda i: (0, i, 0)),
            pl.BlockSpec((_NQ, _RN, _FQ), lambda i: (0, i, 0)),
            pl.BlockSpec((_RN, _FQ), lambda i: (i, 0)),
            pl.BlockSpec((1, _D), lambda i: (0, 0)),
            pl.BlockSpec((_D, _D), lambda i: (0, 0)),
        ],
        out_specs=pl.BlockSpec((_NQ, _RN, _FQ), lambda i: (0, i, 0)),
        out_shape=jax.ShapeDtypeStruct((_NQ, _N, _FQ), jnp.float32),
    )(seg4, hwp, dinv16, b_prev1, W_next)


def _log_softmax(v):
    m = jnp.max(v, axis=1, keepdims=True)
    e = v - m
    return e - jnp.log(jnp.sum(jnp.exp(e), axis=1, keepdims=True))


def _kh_body(xinfo_ref, xtext_ref, xtitle_ref, id_ref, demb_ref,
             tW_ref, tb_ref, iW_ref, ib_ref, fW_ref, fb_ref, w_ref,
             base_ref):
    info = jnp.dot(xinfo_ref[...], tW_ref[...],
                   preferred_element_type=jnp.float32) + tb_ref[...]
    text = jnp.dot(xtext_ref[...], iW_ref[...],
                   preferred_element_type=jnp.float32) + ib_ref[...]
    modals = w_ref[0] * info + w_ref[1] * text
    title = jnp.dot(xtitle_ref[...], fW_ref[...],
                    preferred_element_type=jnp.float32) + fb_ref[...]
    onehot = (id_ref[...] ==
              lax.broadcasted_iota(jnp.int32, (_RB, _NDIR), 1)).astype(jnp.float32)
    director = jnp.dot(onehot, demb_ref[...], preferred_element_type=jnp.float32)
    base_ref[...] = (_log_softmax(modals) + _log_softmax(title)
                     + _log_softmax(director))


def _tc_heads(xinfo, xtext, xtitle, id2, dir_emb,
              text_fc_W, text_fc_b1, image_fc_W, image_fc_b1,
              feat_W, feat_b1, weight):
    return pl.pallas_call(
        _kh_body,
        grid=(_B // _RB,),
        in_specs=[
            pl.BlockSpec((_RB, _IN), lambda i: (i, 0)),
            pl.BlockSpec((_RB, _IN), lambda i: (i, 0)),
            pl.BlockSpec((_RB, _IN), lambda i: (i, 0)),
            pl.BlockSpec((_RB, 1), lambda i: (i, 0)),
            pl.BlockSpec((_NDIR, _D), lambda i: (0, 0)),
            pl.BlockSpec((_IN, _D), lambda i: (0, 0)),
            pl.BlockSpec((1, _D), lambda i: (0, 0)),
            pl.BlockSpec((_IN, _D), lambda i: (0, 0)),
            pl.BlockSpec((1, _D), lambda i: (0, 0)),
            pl.BlockSpec((_IN, _D), lambda i: (0, 0)),
            pl.BlockSpec((1, _D), lambda i: (0, 0)),
            pl.BlockSpec(memory_space=pltpu.SMEM),
        ],
        out_specs=pl.BlockSpec((_RB, _D), lambda i: (i, 0)),
        out_shape=jax.ShapeDtypeStruct((_B, _D), jnp.float32),
    )(xinfo, xtext, xtitle, id2, dir_emb, text_fc_W, text_fc_b1,
      image_fc_W, image_fc_b1, feat_W, feat_b1, weight)


def _kg_body(segg_ref, hwg_ref, dinvg_ref, base_ref, b3_ref,
             cW_ref, cb_ref, rW_ref, rb_ref, oc_ref, orr_ref):
    seg = jnp.concatenate([segg_ref[qq] for qq in range(_NQ)], axis=1)
    hw = jnp.concatenate([hwg_ref[qq] for qq in range(_NQ)], axis=1)
    anchor = dinvg_ref[:, 0:1] * (seg + hw) + b3_ref[...]
    out = anchor + base_ref[...]
    oc_ref[...] = jnp.dot(out, cW_ref[...],
                          preferred_element_type=jnp.float32) + cb_ref[...]
    orr_ref[...] = jnp.dot(out, rW_ref[...],
                           preferred_element_type=jnp.float32) + rb_ref[...]


def _tc_final(segg, hwg, dinvg, base, b31, cls_W, cls_b1, reg_W, reg_b1):
    return pl.pallas_call(
        _kg_body,
        grid=(_B // _RB,),
        in_specs=[
            pl.BlockSpec((_NQ, _RB, _FQ), lambda i: (0, i, 0)),
            pl.BlockSpec((_NQ, _RB, _FQ), lambda i: (0, i, 0)),
            pl.BlockSpec((_RB, _FQ), lambda i: (i, 0)),
            pl.BlockSpec((_RB, _D), lambda i: (i, 0)),
            pl.BlockSpec((1, _D), lambda i: (0, 0)),
            pl.BlockSpec((_D, _OUT), lambda i: (0, 0)),
            pl.BlockSpec((1, _OUT), lambda i: (0, 0)),
            pl.BlockSpec((_D, 1), lambda i: (0, 0)),
            pl.BlockSpec((1, 1), lambda i: (0, 0)),
        ],
        out_specs=[
            pl.BlockSpec((_RB, _OUT), lambda i: (i, 0)),
            pl.BlockSpec((_RB, 1), lambda i: (i, 0)),
        ],
        out_shape=[
            jax.ShapeDtypeStruct((_B, _OUT), jnp.float32),
            jax.ShapeDtypeStruct((_B, 1), jnp.float32),
        ],
    )(segg, hwg, dinvg, base, b31, cls_W, cls_b1, reg_W, reg_b1)


def kernel(id, edge_index, x, index, info_index, text_index, image_index,
           text_fc_W, text_fc_b, image_fc_W, image_fc_b, feat_fc_W, feat_fc_b,
           dir_emb, weight, gcn_W1, gcn_b1, gcn_W2, gcn_b2, gcn_W3, gcn_b3,
           cls_W, cls_b, reg_W, reg_b):
    src = edge_index[0].astype(jnp.int32)
    dst = edge_index[1].astype(jnp.int32)
    npad = _EP - _E
    pad_src = (jnp.arange(npad, dtype=jnp.int32) * 67) % _N
    pad_dst = _N + (jnp.arange(npad, dtype=jnp.int32) % (_SPR - _N))
    src_p = jnp.concatenate([src, pad_src])
    dst3 = jnp.concatenate([dst, pad_dst])
    src4 = jnp.concatenate([src_p + (qq * _N) for qq in range(_NQ)])
    index = index.astype(jnp.int32)
    idx4 = jnp.concatenate([index + (qq * _N) for qq in range(_NQ)])
    id2 = id.astype(jnp.int32)[:, None]

    deg2f, xinfo, xtext, xtitle = _sc_prep(
        dst3, info_index.astype(jnp.int32), text_index.astype(jnp.int32),
        index, x)
    deg2 = deg2f.reshape(_NC, _SPR, _FQ)

    hw1, dinv16 = _tc_first(x, deg2, feat_fc_W, feat_fc_b[None, :], gcn_W1)

    base = _tc_heads(xinfo, xtext, xtitle, id2, dir_emb,
                     text_fc_W, text_fc_b[None, :],
                     image_fc_W, image_fc_b[None, :],
                     feat_fc_W, feat_fc_b[None, :], weight)

    seg1 = _sc_segsum(src4, dst3, hw1.reshape(_NQ * _N, _FQ))
    hw2 = _tc_layer(seg1.reshape(_NQ, _N, _FQ), hw1, dinv16,
                    gcn_b1[None, :], gcn_W2)
    seg2 = _sc_segsum(src4, dst3, hw2.reshape(_NQ * _N, _FQ))
    hw3 = _tc_layer(seg2.reshape(_NQ, _N, _FQ), hw2, dinv16,
                    gcn_b2[None, :], gcn_W3)
    seg3 = _sc_segsum(src4, dst3, hw3.reshape(_NQ * _N, _FQ))

    segg, hwg, dinvg = _sc_final_gather(
        idx4, index, seg3, hw3.reshape(_NQ * _N, _FQ), dinv16)
    oc, orr = _tc_final(segg.reshape(_NQ, _B, _FQ),
                        hwg.reshape(_NQ, _B, _FQ),
                        dinvg, base, gcn_b3[None, :],
                        cls_W, cls_b[None, :], reg_W, reg_b[None, :])
    return (oc, orr)


# speedup vs baseline: 19.1149x; 1.1390x over previous
import functools

import jax
import jax.numpy as jnp
from jax import lax
from jax.experimental import pallas as pl
from jax.experimental.pallas import tpu as pltpu
from jax.experimental.pallas import tpu_sc as plsc

_N = 50000
_E = 800000
_IN = 256
_D = 64
_B = 16384
_NDIR = 120
_OUT = 6

_NC, _NS = 2, 16
_NW = _NC * _NS
_NQ, _FQ = 4, 16
_SPR = 51200
_EP = 802816
_CH = 128
_GRP = 14
_GRPC = 7

_EW = _EP // _NW
_ET = _EP // _NS
_BW = _B // _NW
_BS = _B // _NS

_LAST = (_NS - 1) * 3128


def _mk_mesh():
    return plsc.VectorSubcoreMesh(core_axis_name="c", subcore_axis_name="s",
                                  num_cores=_NC, num_subcores=_NS)


_SC_PARAMS = pltpu.CompilerParams(use_tc_tiling_on_sc=False)


def _fill_rows16(ref, rows, value):
    v = jnp.full((16,), value, jnp.float32)

    @pl.loop(0, rows)
    def _(r):
        ref[r, pl.ds(0, 16)] = v


def _ka_body(dst_hbm, info_hbm, text_hbm, title_hbm, x_hbm,
             deg_out, xinfo_out, xtext_out, xtitle_out,
             *rest):
    didx = rest[0:_GRP]
    gidx = rest[_GRP]
    ones_v = rest[_GRP + 1]
    zbuf = rest[_GRP + 2]
    rows = rest[_GRP + 3]
    deg_sh = rest[_GRP + 4]
    lsem, gsem, ssem = rest[_GRP + 5:_GRP + 8]

    c = lax.axis_index("c")
    s = lax.axis_index("s")
    wid = s * _NC + c

    _fill_rows16(zbuf, _CH, 0.0)
    _fill_rows16(ones_v, _CH, 1.0)
    zcp = [pltpu.async_copy(zbuf, deg_sh.at[pl.ds(s * 3200 + k * _CH, _CH)],
                            lsem) for k in range(25)]
    for cp in zcp:
        cp.wait()
    plsc.subcore_barrier()

    ebase = wid * _EW

    @pl.loop(0, _EW // (_GRP * _CH))
    def _(g):
        goff = ebase + g * (_GRP * _CH)
        loads = [pltpu.async_copy(dst_hbm.at[pl.ds(goff + j * _CH, _CH)],
                                  didx[j], lsem) for j in range(_GRP)]
        for cp in loads:
            cp.wait()
        adds = [pltpu.async_copy(ones_v, deg_sh.at[didx[j]], ssem, add=True)
                for j in range(_GRP)]
        for cp in adds:
            cp.wait()

    plsc.subcore_barrier()
    pltpu.sync_copy(deg_sh.at[pl.ds(s * 3200, 3200)],
                    deg_out.at[pl.ds(c * _SPR + s * 3200, 3200)])

    rbase = wid * _BW
    for idx_hbm, out_hbm in ((info_hbm, xinfo_out), (text_hbm, xtext_out),
                             (title_hbm, xtitle_out)):
        @pl.loop(0, _BW // _CH)
        def _(t):
            off = rbase + t * _CH
            pltpu.sync_copy(idx_hbm.at[pl.ds(off, _CH)], gidx)
            pltpu.async_copy(x_hbm.at[gidx], rows, gsem).wait()
            pltpu.sync_copy(rows, out_hbm.at[pl.ds(off, _CH)])


def _sc_prep(dst3, info_index, text_index, title_index, x):
    kern = pl.kernel(
        _ka_body,
        out_type=[
            jax.ShapeDtypeStruct((_NC * _SPR, _FQ), jnp.float32),
            jax.ShapeDtypeStruct((_B, _IN), jnp.float32),
            jax.ShapeDtypeStruct((_B, _IN), jnp.float32),
            jax.ShapeDtypeStruct((_B, _IN), jnp.float32),
        ],
        mesh=_mk_mesh(),
        compiler_params=_SC_PARAMS,
        scratch_types=(
            [pltpu.VMEM((_CH,), jnp.int32) for _ in range(_GRP)]
            + [pltpu.VMEM((_CH,), jnp.int32),
               pltpu.VMEM((_CH, _FQ), jnp.float32),
               pltpu.VMEM((_CH, _FQ), jnp.float32),
               pltpu.VMEM((_CH, _IN), jnp.float32),
               pltpu.VMEM_SHARED((_SPR, _FQ), jnp.float32),
               pltpu.SemaphoreType.DMA,
               pltpu.SemaphoreType.DMA,
               pltpu.SemaphoreType.DMA]
        ),
    )
    return kern(dst3, info_index, text_index, title_index, x)


def _kc_body(src4_hbm, dst_hbm, hw_hbm, seg_out, *rest):
    sidx_all = rest[0]
    didx = (rest[1:1 + _GRPC], rest[1 + _GRPC:1 + 2 * _GRPC])
    rows = rest[1 + 2 * _GRPC:3 + 2 * _GRPC]
    zbuf = rest[3 + 2 * _GRPC]
    acc = rest[4 + 2 * _GRPC]
    lsem = rest[5 + 2 * _GRPC:7 + 2 * _GRPC]
    gsem = rest[7 + 2 * _GRPC:9 + 2 * _GRPC]
    ssem = rest[9 + 2 * _GRPC:11 + 2 * _GRPC]

    c = lax.axis_index("c")
    s = lax.axis_index("s")
    _fill_rows16(zbuf, _CH, 0.0)

    nhalf = 98
    niter = nhalf // (2 * _GRPC)

    def _fire_didx(row0, gp, p):
        for j in range(_GRPC):
            off = (row0 + gp * _GRPC + j) * _CH
            pltpu.async_copy(dst_hbm.at[pl.ds(off, _CH)], didx[p][j], lsem[p])

    def _drain(sem, srcref, dstref, n):
        for _ in range(n):
            pltpu.make_async_copy(srcref, dstref, sem).wait()

    for qp in range(2):
        q = c * 2 + qp

        zcp = [pltpu.async_copy(zbuf, acc.at[pl.ds(s * 3200 + k * _CH, _CH)],
                                lsem[0]) for k in range(25)]
        for cp in zcp:
            cp.wait()
        plsc.subcore_barrier()

        for half in range(4):
            row0 = s * 392 + half * nhalf
            pltpu.async_copy(src4_hbm.at[pl.ds(q * 6272 + row0, nhalf)],
                             sidx_all, lsem[0]).wait()
            _fire_didx(row0, 0, 0)
            _fire_didx(row0, 1, 1)

            @pl.loop(0, niter)
            def _(G):
                gath0 = [pltpu.async_copy(
                    hw_hbm.at[sidx_all.at[(2 * G) * _GRPC + j]],
                    rows[0].at[j], gsem[0]) for j in range(_GRPC)]
                gath1 = [pltpu.async_copy(
                    hw_hbm.at[sidx_all.at[(2 * G + 1) * _GRPC + j]],
                    rows[1].at[j], gsem[1]) for j in range(_GRPC)]
                _drain(lsem[0], dst_hbm.at[pl.ds(0, _CH)], didx[0][0], _GRPC)
                for cp in gath0:
                    cp.wait()
                adds0 = [pltpu.async_copy(rows[0].at[j], acc.at[didx[0][j]],
                                          ssem[0], add=True)
                         for j in range(_GRPC)]
                _drain(lsem[1], dst_hbm.at[pl.ds(0, _CH)], didx[1][0], _GRPC)
                for cp in gath1:
                    cp.wait()
                adds1 = [pltpu.async_copy(rows[1].at[j], acc.at[didx[1][j]],
                                          ssem[1], add=True)
                         for j in range(_GRPC)]
                for cp in adds0:
                    cp.wait()
                for cp in adds1:
                    cp.wait()

                @pl.when(G < niter - 1)
                def _():
                    _fire_didx(row0, 2 * G + 2, 0)
                    _fire_didx(row0, 2 * G + 3, 1)

        plsc.subcore_barrier()

        @pl.when(s < _NS - 1)
        def _():
            pltpu.sync_copy(acc.at[pl.ds(s * 3128, 3128)],
                            seg_out.at[pl.ds(q * _N + s * 3128, 3128)])

        @pl.when(s == _NS - 1)
        def _():
            pltpu.sync_copy(acc.at[pl.ds(_LAST, _N - _LAST)],
                            seg_out.at[pl.ds(q * _N + _LAST, _N - _LAST)])

        plsc.subcore_barrier()


def _sc_segsum(src4, dst3, hw4):
    kern = pl.kernel(
        _kc_body,
        out_type=jax.ShapeDtypeStruct((_NQ * _N, _FQ), jnp.float32),
        mesh=_mk_mesh(),
        compiler_params=_SC_PARAMS,
        scratch_types=(
            [pltpu.VMEM((98, _CH), jnp.int32)]
            + [pltpu.VMEM((_CH,), jnp.int32) for _ in range(2 * _GRPC)]
            + [pltpu.VMEM((_GRPC, _CH, _FQ), jnp.float32),
               pltpu.VMEM((_GRPC, _CH, _FQ), jnp.float32),
               pltpu.VMEM((_CH, _FQ), jnp.float32),
               pltpu.VMEM_SHARED((_SPR, _FQ), jnp.float32)]
            + [pltpu.SemaphoreType.DMA for _ in range(6)]
        ),
    )
    return kern(src4, dst3, hw4)


def _kf_body(idx4_hbm, idx_hbm, seg_hbm, hw_hbm, dinv_hbm,
             segg_out, hwg_out, dinvg_out, *rest):
    gidx, rows_a, rows_b = rest[0:3]
    gsem = rest[3]

    c = lax.axis_index("c")
    s = lax.axis_index("s")
    rbase = s * _BS

    for qp in range(2):
        q = c * 2 + qp

        @pl.loop(0, _BS // _CH)
        def _(t):
            off = rbase + t * _CH
            pltpu.sync_copy(idx4_hbm.at[pl.ds(q * _B + off, _CH)], gidx)
            pltpu.async_copy(seg_hbm.at[gidx], rows_a, gsem).wait()
            pltpu.sync_copy(rows_a, segg_out.at[pl.ds(q * _B + off, _CH)])
            pltpu.async_copy(hw_hbm.at[gidx], rows_b, gsem).wait()
            pltpu.sync_copy(rows_b, hwg_out.at[pl.ds(q * _B + off, _CH)])

    @pl.when(c == 0)
    def _():
        @pl.loop(0, _BS // _CH)
        def _(t):
            off = rbase + t * _CH
            pltpu.sync_copy(idx_hbm.at[pl.ds(off, _CH)], gidx)
            pltpu.async_copy(dinv_hbm.at[gidx], rows_a, gsem).wait()
            pltpu.sync_copy(rows_a, dinvg_out.at[pl.ds(off, _CH)])


def _sc_final_gather(idx4, index, seg4, hw4, dinv16):
    kern = pl.kernel(
        _kf_body,
        out_type=[
            jax.ShapeDtypeStruct((_NQ * _B, _FQ), jnp.float32),
            jax.ShapeDtypeStruct((_NQ * _B, _FQ), jnp.float32),
            jax.ShapeDtypeStruct((_B, _FQ), jnp.float32),
        ],
        mesh=_mk_mesh(),
        compiler_params=_SC_PARAMS,
        scratch_types=[
            pltpu.VMEM((_CH,), jnp.int32),
            pltpu.VMEM((_CH, _FQ), jnp.float32),
            pltpu.VMEM((_CH, _FQ), jnp.float32),
            pltpu.SemaphoreType.DMA,
        ],
    )
    return kern(idx4, index, seg4, hw4, dinv16)


_RN = 1024
_RB = 1024


def _kb_body(x_ref, deg_ref, fw_ref, fb_ref, w1q_ref, hw_ref, dinv_ref):
    dsum = deg_ref[0, :, 0:1] + deg_ref[1, :, 0:1] + 1.0
    dinv = lax.rsqrt(dsum)
    h = jnp.dot(x_ref[...], fw_ref[...],
                preferred_element_type=jnp.float32) + fb_ref[...]
    for qq in range(_NQ):
        hw_ref[qq] = jnp.dot(h, w1q_ref[qq],
                             preferred_element_type=jnp.float32) * dinv
    dinv_ref[...] = jnp.broadcast_to(dinv, (_RN, _FQ))


def _tc_first(x, deg2, feat_W, feat_b1, gcn_W1):
    return pl.pallas_call(
        _kb_body,
        grid=(pl.cdiv(_N, _RN),),
        in_specs=[
            pl.BlockSpec((_RN, _IN), lambda i: (i, 0)),
            pl.BlockSpec((_NC, _RN, _FQ), lambda i: (0, i, 0)),
            pl.BlockSpec((_IN, _D), lambda i: (0, 0)),
            pl.BlockSpec((1, _D), lambda i: (0, 0)),
            pl.BlockSpec((_NQ, _D, _FQ), lambda i: (0, 0, 0)),
        ],
        out_specs=[
            pl.BlockSpec((_NQ, _RN, _FQ), lambda i: (0, i, 0)),
            pl.BlockSpec((_RN, _FQ), lambda i: (i, 0)),
        ],
        out_shape=[
            jax.ShapeDtypeStruct((_NQ, _N, _FQ), jnp.float32),
            jax.ShapeDtypeStruct((_N, _FQ), jnp.float32),
        ],
    )(x, deg2, feat_W, feat_b1, gcn_W1)


def _kd_body(seg_ref, hw_ref, dinv_ref, b_ref, wrow_ref, out_ref):
    dinv = dinv_ref[:, 0:1]
    hc = [jnp.maximum(dinv * (seg_ref[qq] + hw_ref[qq]) + b_ref[qq], 0.0)
          for qq in range(_NQ)]
    hwn = sum(jnp.dot(hc[qq], wrow_ref[qq],
                      preferred_element_type=jnp.float32) for qq in range(_NQ))
    hwn = hwn * dinv
    for qq in range(_NQ):
        out_ref[qq] = hwn[:, qq * _FQ:(qq + 1) * _FQ]


def _tc_layer(seg4, hwp, dinv16, b_prev1, W_next):
    return pl.pallas_call(
        _kd_body,
        grid=(pl.cdiv(_N, _RN),),
        in_specs=[
            pl.BlockSpec((_NQ, _RN, _FQ), lambda i: (0, i, 0)),
            pl.BlockSpec((_NQ, _RN, _FQ), lambda i: (0, i, 0)),
            pl.BlockSpec((_RN, _FQ), lambda i: (i, 0)),
            pl.BlockSpec((_NQ, 1, _FQ), lambda i: (0, 0, 0)),
            pl.BlockSpec((_NQ, _FQ, _D), lambda i: (0, 0, 0)),
        ],
        out_specs=pl.BlockSpec((_NQ, _RN, _FQ), lambda i: (0, i, 0)),
        out_shape=jax.ShapeDtypeStruct((_NQ, _N, _FQ), jnp.float32),
    )(seg4, hwp, dinv16, b_prev1, W_next)


def _log_softmax(v):
    m = jnp.max(v, axis=1, keepdims=True)
    e = v - m
    return e - jnp.log(jnp.sum(jnp.exp(e), axis=1, keepdims=True))


def _kh_body(xinfo_ref, xtext_ref, xtitle_ref, id_ref, demb_ref,
             tW_ref, tb_ref, iW_ref, ib_ref, fW_ref, fb_ref, w_ref,
             base_ref):
    info = jnp.dot(xinfo_ref[...], tW_ref[...],
                   preferred_element_type=jnp.float32) + tb_ref[...]
    text = jnp.dot(xtext_ref[...], iW_ref[...],
                   preferred_element_type=jnp.float32) + ib_ref[...]
    modals = w_ref[0] * info + w_ref[1] * text
    title = jnp.dot(xtitle_ref[...], fW_ref[...],
                    preferred_element_type=jnp.float32) + fb_ref[...]
    onehot = (id_ref[...] ==
              lax.broadcasted_iota(jnp.int32, (_RB, _NDIR), 1)).astype(jnp.float32)
    director = jnp.dot(onehot, demb_ref[...], preferred_element_type=jnp.float32)
    base_ref[...] = (_log_softmax(modals) + _log_softmax(title)
                     + _log_softmax(director))


def _tc_heads(xinfo, xtext, xtitle, id2, dir_emb,
              text_fc_W, text_fc_b1, image_fc_W, image_fc_b1,
              feat_W, feat_b1, weight):
    return pl.pallas_call(
        _kh_body,
        grid=(_B // _RB,),
        in_specs=[
            pl.BlockSpec((_RB, _IN), lambda i: (i, 0)),
            pl.BlockSpec((_RB, _IN), lambda i: (i, 0)),
            pl.BlockSpec((_RB, _IN), lambda i: (i, 0)),
            pl.BlockSpec((_RB, 1), lambda i: (i, 0)),
            pl.BlockSpec((_NDIR, _D), lambda i: (0, 0)),
            pl.BlockSpec((_IN, _D), lambda i: (0, 0)),
            pl.BlockSpec((1, _D), lambda i: (0, 0)),
            pl.BlockSpec((_IN, _D), lambda i: (0, 0)),
            pl.BlockSpec((1, _D), lambda i: (0, 0)),
            pl.BlockSpec((_IN, _D), lambda i: (0, 0)),
            pl.BlockSpec((1, _D), lambda i: (0, 0)),
            pl.BlockSpec(memory_space=pltpu.SMEM),
        ],
        out_specs=pl.BlockSpec((_RB, _D), lambda i: (i, 0)),
        out_shape=jax.ShapeDtypeStruct((_B, _D), jnp.float32),
    )(xinfo, xtext, xtitle, id2, dir_emb, text_fc_W, text_fc_b1,
      image_fc_W, image_fc_b1, feat_W, feat_b1, weight)


def _kg_body(segg_ref, hwg_ref, dinvg_ref, base_ref, b3_ref,
             cW_ref, cWr_ref, cb_ref, rW_ref, rWr_ref, rb_ref,
             oc_ref, orr_ref):
    dinv = dinvg_ref[:, 0:1]
    anc = [dinv * (segg_ref[qq] + hwg_ref[qq]) + b3_ref[qq]
           for qq in range(_NQ)]
    oc = jnp.dot(base_ref[...], cW_ref[...],
                 preferred_element_type=jnp.float32) + cb_ref[...]
    orr = jnp.dot(base_ref[...], rW_ref[...],
                  preferred_element_type=jnp.float32) + rb_ref[...]
    for qq in range(_NQ):
        oc = oc + jnp.dot(anc[qq], cWr_ref[qq],
                          preferred_element_type=jnp.float32)
        orr = orr + jnp.dot(anc[qq], rWr_ref[qq],
                            preferred_element_type=jnp.float32)
    oc_ref[...] = oc
    orr_ref[...] = orr


def _tc_final(segg, hwg, dinvg, base, b3q, cls_W, cls_Wr, cls_b1,
              reg_W, reg_Wr, reg_b1):
    return pl.pallas_call(
        _kg_body,
        grid=(_B // _RB,),
        in_specs=[
            pl.BlockSpec((_NQ, _RB, _FQ), lambda i: (0, i, 0)),
            pl.BlockSpec((_NQ, _RB, _FQ), lambda i: (0, i, 0)),
            pl.BlockSpec((_RB, _FQ), lambda i: (i, 0)),
            pl.BlockSpec((_RB, _D), lambda i: (i, 0)),
            pl.BlockSpec((_NQ, 1, _FQ), lambda i: (0, 0, 0)),
            pl.BlockSpec((_D, _OUT), lambda i: (0, 0)),
            pl.BlockSpec((_NQ, _FQ, _OUT), lambda i: (0, 0, 0)),
            pl.BlockSpec((1, _OUT), lambda i: (0, 0)),
            pl.BlockSpec((_D, 1), lambda i: (0, 0)),
            pl.BlockSpec((_NQ, _FQ, 1), lambda i: (0, 0, 0)),
            pl.BlockSpec((1, 1), lambda i: (0, 0)),
        ],
        out_specs=[
            pl.BlockSpec((_RB, _OUT), lambda i: (i, 0)),
            pl.BlockSpec((_RB, 1), lambda i: (i, 0)),
        ],
        out_shape=[
            jax.ShapeDtypeStruct((_B, _OUT), jnp.float32),
            jax.ShapeDtypeStruct((_B, 1), jnp.float32),
        ],
    )(segg, hwg, dinvg, base, b3q, cls_W, cls_Wr, cls_b1,
      reg_W, reg_Wr, reg_b1)


def kernel(id, edge_index, x, index, info_index, text_index, image_index,
           text_fc_W, text_fc_b, image_fc_W, image_fc_b, feat_fc_W, feat_fc_b,
           dir_emb, weight, gcn_W1, gcn_b1, gcn_W2, gcn_b2, gcn_W3, gcn_b3,
           cls_W, cls_b, reg_W, reg_b):
    src = edge_index[0].astype(jnp.int32)
    dst = edge_index[1].astype(jnp.int32)
    npad = _EP - _E
    pad_src = (jnp.arange(npad, dtype=jnp.int32) * 67) % _N
    pad_dst = _N + (jnp.arange(npad, dtype=jnp.int32) % (_SPR - _N))
    src_p = jnp.concatenate([src, pad_src])
    dst3 = jnp.concatenate([dst, pad_dst])
    src4 = jnp.concatenate(
        [src_p + (qq * _N) for qq in range(_NQ)]).reshape(_NQ * _EP // _CH, _CH)
    index = index.astype(jnp.int32)
    idx4 = jnp.concatenate([index + (qq * _N) for qq in range(_NQ)])
    id2 = id.astype(jnp.int32)[:, None]

    deg2f, xinfo, xtext, xtitle = _sc_prep(
        dst3, info_index.astype(jnp.int32), text_index.astype(jnp.int32),
        index, x)
    deg2 = deg2f.reshape(_NC, _SPR, _FQ)

    w1q = gcn_W1.T.reshape(_NQ, _FQ, _D).transpose(0, 2, 1)
    hw1, dinv16 = _tc_first(x, deg2, feat_fc_W, feat_fc_b[None, :], w1q)

    base = _tc_heads(xinfo, xtext, xtitle, id2, dir_emb,
                     text_fc_W, text_fc_b[None, :],
                     image_fc_W, image_fc_b[None, :],
                     feat_fc_W, feat_fc_b[None, :], weight)

    w2r = gcn_W2.reshape(_NQ, _FQ, _D)
    w3r = gcn_W3.reshape(_NQ, _FQ, _D)
    b1q = gcn_b1.reshape(_NQ, 1, _FQ)
    b2q = gcn_b2.reshape(_NQ, 1, _FQ)
    seg1 = _sc_segsum(src4, dst3, hw1.reshape(_NQ * _N, _FQ))
    hw2 = _tc_layer(seg1.reshape(_NQ, _N, _FQ), hw1, dinv16, b1q, w2r)
    seg2 = _sc_segsum(src4, dst3, hw2.reshape(_NQ * _N, _FQ))
    hw3 = _tc_layer(seg2.reshape(_NQ, _N, _FQ), hw2, dinv16, b2q, w3r)
    seg3 = _sc_segsum(src4, dst3, hw3.reshape(_NQ * _N, _FQ))

    segg, hwg, dinvg = _sc_final_gather(
        idx4, index, seg3, hw3.reshape(_NQ * _N, _FQ), dinv16)
    oc, orr = _tc_final(segg.reshape(_NQ, _B, _FQ),
                        hwg.reshape(_NQ, _B, _FQ),
                        dinvg, base, gcn_b3.reshape(_NQ, 1, _FQ),
                        cls_W, cls_W.reshape(_NQ, _FQ, _OUT), cls_b[None, :],
                        reg_W, reg_W.reshape(_NQ, _FQ, 1), reg_b[None, :])
    return (oc, orr)


# speedup vs baseline: 23.9090x; 1.2508x over previous
import functools

import jax
import jax.numpy as jnp
from jax import lax
from jax.experimental import pallas as pl
from jax.experimental.pallas import tpu as pltpu
from jax.experimental.pallas import tpu_sc as plsc

_N = 50000
_E = 800000
_IN = 256
_D = 64
_B = 16384
_NDIR = 120
_OUT = 6

_NC, _NS = 2, 16
_NW = _NC * _NS
_NQ, _FQ = 4, 16
_SPR = 51200
_EP = 802816
_CH = 128
_GRP = 14
_GRPC = 7

_EW = _EP // _NW
_ET = _EP // _NS
_BW = _B // _NW
_BS = _B // _NS

_LAST = (_NS - 1) * 3128


def _mk_mesh():
    return plsc.VectorSubcoreMesh(core_axis_name="c", subcore_axis_name="s",
                                  num_cores=_NC, num_subcores=_NS)


_SC_PARAMS = pltpu.CompilerParams(use_tc_tiling_on_sc=False)


def _fill_rows16(ref, rows, value):
    v = jnp.full((16,), value, jnp.float32)

    @pl.loop(0, rows)
    def _(r):
        ref[r, pl.ds(0, 16)] = v


def _ka_body(dst_hbm, info_hbm, text_hbm, title_hbm, x_hbm,
             deg_out, xinfo_out, xtext_out, xtitle_out,
             *rest):
    didx = rest[0:_GRP]
    gidx = rest[_GRP]
    ones_v = rest[_GRP + 1]
    zbuf = rest[_GRP + 2]
    rows = rest[_GRP + 3]
    deg_sh = rest[_GRP + 4]
    lsem, gsem, ssem = rest[_GRP + 5:_GRP + 8]

    c = lax.axis_index("c")
    s = lax.axis_index("s")
    wid = s * _NC + c

    _fill_rows16(zbuf, _CH, 0.0)
    _fill_rows16(ones_v, _CH, 1.0)
    zcp = [pltpu.async_copy(zbuf, deg_sh.at[pl.ds(s * 3200 + k * _CH, _CH)],
                            lsem) for k in range(25)]
    for cp in zcp:
        cp.wait()
    plsc.subcore_barrier()

    ebase = wid * _EW

    @pl.loop(0, _EW // (_GRP * _CH))
    def _(g):
        goff = ebase + g * (_GRP * _CH)
        loads = [pltpu.async_copy(dst_hbm.at[pl.ds(goff + j * _CH, _CH)],
                                  didx[j], lsem) for j in range(_GRP)]
        for cp in loads:
            cp.wait()
        adds = [pltpu.async_copy(ones_v, deg_sh.at[didx[j]], ssem, add=True)
                for j in range(_GRP)]
        for cp in adds:
            cp.wait()

    plsc.subcore_barrier()
    pltpu.sync_copy(deg_sh.at[pl.ds(s * 3200, 3200)],
                    deg_out.at[pl.ds(c * _SPR + s * 3200, 3200)])

    rbase = wid * _BW
    for idx_hbm, out_hbm in ((info_hbm, xinfo_out), (text_hbm, xtext_out),
                             (title_hbm, xtitle_out)):
        @pl.loop(0, _BW // _CH)
        def _(t):
            off = rbase + t * _CH
            pltpu.sync_copy(idx_hbm.at[pl.ds(off, _CH)], gidx)
            pltpu.async_copy(x_hbm.at[gidx], rows, gsem).wait()
            pltpu.sync_copy(rows, out_hbm.at[pl.ds(off, _CH)])


def _sc_prep(dst3, info_index, text_index, title_index, x):
    kern = pl.kernel(
        _ka_body,
        out_type=[
            jax.ShapeDtypeStruct((_NC * _SPR, _FQ), jnp.float32),
            jax.ShapeDtypeStruct((_B, _IN), jnp.float32),
            jax.ShapeDtypeStruct((_B, _IN), jnp.float32),
            jax.ShapeDtypeStruct((_B, _IN), jnp.float32),
        ],
        mesh=_mk_mesh(),
        compiler_params=_SC_PARAMS,
        scratch_types=(
            [pltpu.VMEM((_CH,), jnp.int32) for _ in range(_GRP)]
            + [pltpu.VMEM((_CH,), jnp.int32),
               pltpu.VMEM((_CH, _FQ), jnp.float32),
               pltpu.VMEM((_CH, _FQ), jnp.float32),
               pltpu.VMEM((_CH, _IN), jnp.float32),
               pltpu.VMEM_SHARED((_SPR, _FQ), jnp.float32),
               pltpu.SemaphoreType.DMA,
               pltpu.SemaphoreType.DMA,
               pltpu.SemaphoreType.DMA]
        ),
    )
    return kern(dst3, info_index, text_index, title_index, x)


def _kc_body(src4_hbm, dst_hbm, hw_hbm, seg_out, *rest):
    sidx_all = rest[0]
    didx = (rest[1:1 + _GRPC], rest[1 + _GRPC:1 + 2 * _GRPC])
    rows = rest[1 + 2 * _GRPC:3 + 2 * _GRPC]
    zbuf = rest[3 + 2 * _GRPC]
    acc = rest[4 + 2 * _GRPC]
    lsem = rest[5 + 2 * _GRPC:7 + 2 * _GRPC]
    gsem = rest[7 + 2 * _GRPC:9 + 2 * _GRPC]
    ssem = rest[9 + 2 * _GRPC:11 + 2 * _GRPC]

    c = lax.axis_index("c")
    s = lax.axis_index("s")
    _fill_rows16(zbuf, _CH, 0.0)

    nhalf = 98
    niter = nhalf // (2 * _GRPC)

    def _fire_didx(row0, gp, p):
        for j in range(_GRPC):
            off = (row0 + gp * _GRPC + j) * _CH
            pltpu.async_copy(dst_hbm.at[pl.ds(off, _CH)], didx[p][j], lsem[p])

    def _drain(sem, srcref, dstref, n):
        for _ in range(n):
            pltpu.make_async_copy(srcref, dstref, sem).wait()

    for qp in range(2):
        q = c * 2 + qp

        zcp = [pltpu.async_copy(zbuf, acc.at[pl.ds(s * 3200 + k * _CH, _CH)],
                                lsem[0]) for k in range(25)]
        for cp in zcp:
            cp.wait()
        plsc.subcore_barrier()

        for half in range(4):
            row0 = s * 392 + half * nhalf
            pltpu.async_copy(src4_hbm.at[pl.ds(q * 6272 + row0, nhalf)],
                             sidx_all, lsem[0]).wait()
            _fire_didx(row0, 0, 0)
            _fire_didx(row0, 1, 1)

            @pl.loop(0, niter)
            def _(G):
                gath0 = [pltpu.async_copy(
                    hw_hbm.at[sidx_all.at[(2 * G) * _GRPC + j]],
                    rows[0].at[j], gsem[0]) for j in range(_GRPC)]
                gath1 = [pltpu.async_copy(
                    hw_hbm.at[sidx_all.at[(2 * G + 1) * _GRPC + j]],
                    rows[1].at[j], gsem[1]) for j in range(_GRPC)]
                _drain(lsem[0], dst_hbm.at[pl.ds(0, _CH)], didx[0][0], _GRPC)
                for cp in gath0:
                    cp.wait()
                adds0 = [pltpu.async_copy(rows[0].at[j], acc.at[didx[0][j]],
                                          ssem[0], add=True)
                         for j in range(_GRPC)]
                _drain(lsem[1], dst_hbm.at[pl.ds(0, _CH)], didx[1][0], _GRPC)
                for cp in gath1:
                    cp.wait()
                adds1 = [pltpu.async_copy(rows[1].at[j], acc.at[didx[1][j]],
                                          ssem[1], add=True)
                         for j in range(_GRPC)]
                for cp in adds0:
                    cp.wait()
                for cp in adds1:
                    cp.wait()

                @pl.when(G < niter - 1)
                def _():
                    _fire_didx(row0, 2 * G + 2, 0)
                    _fire_didx(row0, 2 * G + 3, 1)

        plsc.subcore_barrier()

        @pl.when(s < _NS - 1)
        def _():
            pltpu.sync_copy(acc.at[pl.ds(s * 3128, 3128)],
                            seg_out.at[pl.ds(q * _N + s * 3128, 3128)])

        @pl.when(s == _NS - 1)
        def _():
            pltpu.sync_copy(acc.at[pl.ds(_LAST, _N - _LAST)],
                            seg_out.at[pl.ds(q * _N + _LAST, _N - _LAST)])

        plsc.subcore_barrier()


def _sc_segsum(src4, dst3, hw4):
    kern = pl.kernel(
        _kc_body,
        out_type=jax.ShapeDtypeStruct((_NQ * _N, _FQ), jnp.float32),
        mesh=_mk_mesh(),
        compiler_params=_SC_PARAMS,
        scratch_types=(
            [pltpu.VMEM((98, _CH), jnp.int32)]
            + [pltpu.VMEM((_CH,), jnp.int32) for _ in range(2 * _GRPC)]
            + [pltpu.VMEM((_GRPC, _CH, _FQ), jnp.float32),
               pltpu.VMEM((_GRPC, _CH, _FQ), jnp.float32),
               pltpu.VMEM((_CH, _FQ), jnp.float32),
               pltpu.VMEM_SHARED((_SPR, _FQ), jnp.float32)]
            + [pltpu.SemaphoreType.DMA for _ in range(6)]
        ),
    )
    return kern(src4, dst3, hw4)


def _kf_body(idx4_hbm, idx_hbm, seg_hbm, hw_hbm, dinv_hbm,
             segg_out, hwg_out, dinvg_out, *rest):
    gidx, rows_a, rows_b = rest[0:3]
    gsem = rest[3]

    c = lax.axis_index("c")
    s = lax.axis_index("s")
    rbase = s * _BS

    for qp in range(2):
        q = c * 2 + qp

        @pl.loop(0, _BS // _CH)
        def _(t):
            off = rbase + t * _CH
            pltpu.sync_copy(idx4_hbm.at[pl.ds(q * _B + off, _CH)], gidx)
            pltpu.async_copy(seg_hbm.at[gidx], rows_a, gsem).wait()
            pltpu.sync_copy(rows_a, segg_out.at[pl.ds(q * _B + off, _CH)])
            pltpu.async_copy(hw_hbm.at[gidx], rows_b, gsem).wait()
            pltpu.sync_copy(rows_b, hwg_out.at[pl.ds(q * _B + off, _CH)])

    @pl.when(c == 0)
    def _():
        @pl.loop(0, _BS // _CH)
        def _(t):
            off = rbase + t * _CH
            pltpu.sync_copy(idx_hbm.at[pl.ds(off, _CH)], gidx)
            pltpu.async_copy(dinv_hbm.at[gidx], rows_a, gsem).wait()
            pltpu.sync_copy(rows_a, dinvg_out.at[pl.ds(off, _CH)])


def _sc_final_gather(idx4, index, seg4, hw4, dinv16):
    kern = pl.kernel(
        _kf_body,
        out_type=[
            jax.ShapeDtypeStruct((_NQ * _B, _FQ), jnp.float32),
            jax.ShapeDtypeStruct((_NQ * _B, _FQ), jnp.float32),
            jax.ShapeDtypeStruct((_B, _FQ), jnp.float32),
        ],
        mesh=_mk_mesh(),
        compiler_params=_SC_PARAMS,
        scratch_types=[
            pltpu.VMEM((_CH,), jnp.int32),
            pltpu.VMEM((_CH, _FQ), jnp.float32),
            pltpu.VMEM((_CH, _FQ), jnp.float32),
            pltpu.SemaphoreType.DMA,
        ],
    )
    return kern(idx4, index, seg4, hw4, dinv16)


_RN = 1024
_RB = 1024


def _kb_body(x_ref, deg_ref, fw_ref, fb_ref, w1q_ref, hw_ref, dinv_ref):
    dsum = deg_ref[0, :, 0:1] + deg_ref[1, :, 0:1] + 1.0
    dinv = lax.rsqrt(dsum)
    h = jnp.dot(x_ref[...], fw_ref[...],
                preferred_element_type=jnp.float32) + fb_ref[...]
    for qq in range(_NQ):
        hw_ref[qq] = jnp.dot(h, w1q_ref[qq],
                             preferred_element_type=jnp.float32) * dinv
    dinv_ref[...] = jnp.broadcast_to(dinv, (_RN, _FQ))


def _tc_first(x, deg2, feat_W, feat_b1, gcn_W1):
    return pl.pallas_call(
        _kb_body,
        grid=(pl.cdiv(_N, _RN),),
        in_specs=[
            pl.BlockSpec((_RN, _IN), lambda i: (i, 0)),
            pl.BlockSpec((_NC, _RN, _FQ), lambda i: (0, i, 0)),
            pl.BlockSpec((_IN, _D), lambda i: (0, 0)),
            pl.BlockSpec((1, _D), lambda i: (0, 0)),
            pl.BlockSpec((_NQ, _D, _FQ), lambda i: (0, 0, 0)),
        ],
        out_specs=[
            pl.BlockSpec((_NQ, _RN, _FQ), lambda i: (0, i, 0)),
            pl.BlockSpec((_RN, _FQ), lambda i: (i, 0)),
        ],
        out_shape=[
            jax.ShapeDtypeStruct((_NQ, _N, _FQ), jnp.float32),
            jax.ShapeDtypeStruct((_N, _FQ), jnp.float32),
        ],
    )(x, deg2, feat_W, feat_b1, gcn_W1)


def _kd_body(seg_ref, hw_ref, dinvf_ref, b_ref, kw_ref, out_ref):
    dinv = dinvf_ref[...]
    hc = [jnp.maximum(dinv * (seg_ref[qq] + hw_ref[qq]) + b_ref[qq], 0.0)
          for qq in range(_NQ)]
    for qo in range(_NQ):
        hwn = sum(jnp.dot(hc[qi], kw_ref[qi, qo],
                          preferred_element_type=jnp.float32)
                  for qi in range(_NQ))
        out_ref[qo] = hwn * dinv


def _tc_layer(seg4, hwp, dinvf, b_tile, kron_W):
    return pl.pallas_call(
        _kd_body,
        grid=(pl.cdiv(_N, _RN),),
        in_specs=[
            pl.BlockSpec((_NQ, _RN // 8, 8 * _FQ), lambda i: (0, i, 0)),
            pl.BlockSpec((_NQ, _RN // 8, 8 * _FQ), lambda i: (0, i, 0)),
            pl.BlockSpec((_RN // 8, 8 * _FQ), lambda i: (i, 0)),
            pl.BlockSpec((_NQ, 1, 8 * _FQ), lambda i: (0, 0, 0)),
            pl.BlockSpec((_NQ, _NQ, 8 * _FQ, 8 * _FQ), lambda i: (0, 0, 0, 0)),
        ],
        out_specs=pl.BlockSpec((_NQ, _RN // 8, 8 * _FQ), lambda i: (0, i, 0)),
        out_shape=jax.ShapeDtypeStruct((_NQ, _N // 8, 8 * _FQ), jnp.float32),
    )(seg4, hwp, dinvf, b_tile, kron_W)


def _log_softmax(v):
    m = jnp.max(v, axis=1, keepdims=True)
    e = v - m
    return e - jnp.log(jnp.sum(jnp.exp(e), axis=1, keepdims=True))


def _kh_body(xinfo_ref, xtext_ref, xtitle_ref, id_ref, demb_ref,
             tW_ref, tb_ref, iW_ref, ib_ref, fW_ref, fb_ref, w_ref,
             base_ref):
    info = jnp.dot(xinfo_ref[...], tW_ref[...],
                   preferred_element_type=jnp.float32) + tb_ref[...]
    text = jnp.dot(xtext_ref[...], iW_ref[...],
                   preferred_element_type=jnp.float32) + ib_ref[...]
    modals = w_ref[0] * info + w_ref[1] * text
    title = jnp.dot(xtitle_ref[...], fW_ref[...],
                    preferred_element_type=jnp.float32) + fb_ref[...]
    onehot = (id_ref[...] ==
              lax.broadcasted_iota(jnp.int32, (_RB, _NDIR), 1)).astype(jnp.float32)
    director = jnp.dot(onehot, demb_ref[...], preferred_element_type=jnp.float32)
    base_ref[...] = (_log_softmax(modals) + _log_softmax(title)
                     + _log_softmax(director))


def _tc_heads(xinfo, xtext, xtitle, id2, dir_emb,
              text_fc_W, text_fc_b1, image_fc_W, image_fc_b1,
              feat_W, feat_b1, weight):
    return pl.pallas_call(
        _kh_body,
        grid=(_B // _RB,),
        in_specs=[
            pl.BlockSpec((_RB, _IN), lambda i: (i, 0)),
            pl.BlockSpec((_RB, _IN), lambda i: (i, 0)),
            pl.BlockSpec((_RB, _IN), lambda i: (i, 0)),
            pl.BlockSpec((_RB, 1), lambda i: (i, 0)),
            pl.BlockSpec((_NDIR, _D), lambda i: (0, 0)),
            pl.BlockSpec((_IN, _D), lambda i: (0, 0)),
            pl.BlockSpec((1, _D), lambda i: (0, 0)),
            pl.BlockSpec((_IN, _D), lambda i: (0, 0)),
            pl.BlockSpec((1, _D), lambda i: (0, 0)),
            pl.BlockSpec((_IN, _D), lambda i: (0, 0)),
            pl.BlockSpec((1, _D), lambda i: (0, 0)),
            pl.BlockSpec(memory_space=pltpu.SMEM),
        ],
        out_specs=pl.BlockSpec((_RB, _D), lambda i: (i, 0)),
        out_shape=jax.ShapeDtypeStruct((_B, _D), jnp.float32),
    )(xinfo, xtext, xtitle, id2, dir_emb, text_fc_W, text_fc_b1,
      image_fc_W, image_fc_b1, feat_W, feat_b1, weight)


def _kg_body(segg_ref, hwg_ref, dinvg_ref, base_ref, b3_ref,
             cW_ref, cWr_ref, cb_ref, rW_ref, rWr_ref, rb_ref,
             oc_ref, orr_ref):
    dinv = dinvg_ref[:, 0:1]
    anc = [dinv * (segg_ref[qq] + hwg_ref[qq]) + b3_ref[qq]
           for qq in range(_NQ)]
    oc = jnp.dot(base_ref[...], cW_ref[...],
                 preferred_element_type=jnp.float32) + cb_ref[...]
    orr = jnp.dot(base_ref[...], rW_ref[...],
                  preferred_element_type=jnp.float32) + rb_ref[...]
    for qq in range(_NQ):
        oc = oc + jnp.dot(anc[qq], cWr_ref[qq],
                          preferred_element_type=jnp.float32)
        orr = orr + jnp.dot(anc[qq], rWr_ref[qq],
                            preferred_element_type=jnp.float32)
    oc_ref[...] = oc
    orr_ref[...] = orr


def _tc_final(segg, hwg, dinvg, base, b3q, cls_W, cls_Wr, cls_b1,
              reg_W, reg_Wr, reg_b1):
    return pl.pallas_call(
        _kg_body,
        grid=(_B // _RB,),
        in_specs=[
            pl.BlockSpec((_NQ, _RB, _FQ), lambda i: (0, i, 0)),
            pl.BlockSpec((_NQ, _RB, _FQ), lambda i: (0, i, 0)),
            pl.BlockSpec((_RB, _FQ), lambda i: (i, 0)),
            pl.BlockSpec((_RB, _D), lambda i: (i, 0)),
            pl.BlockSpec((_NQ, 1, _FQ), lambda i: (0, 0, 0)),
            pl.BlockSpec((_D, _OUT), lambda i: (0, 0)),
            pl.BlockSpec((_NQ, _FQ, _OUT), lambda i: (0, 0, 0)),
            pl.BlockSpec((1, _OUT), lambda i: (0, 0)),
            pl.BlockSpec((_D, 1), lambda i: (0, 0)),
            pl.BlockSpec((_NQ, _FQ, 1), lambda i: (0, 0, 0)),
            pl.BlockSpec((1, 1), lambda i: (0, 0)),
        ],
        out_specs=[
            pl.BlockSpec((_RB, _OUT), lambda i: (i, 0)),
            pl.BlockSpec((_RB, 1), lambda i: (i, 0)),
        ],
        out_shape=[
            jax.ShapeDtypeStruct((_B, _OUT), jnp.float32),
            jax.ShapeDtypeStruct((_B, 1), jnp.float32),
        ],
    )(segg, hwg, dinvg, base, b3q, cls_W, cls_Wr, cls_b1,
      reg_W, reg_Wr, reg_b1)


def kernel(id, edge_index, x, index, info_index, text_index, image_index,
           text_fc_W, text_fc_b, image_fc_W, image_fc_b, feat_fc_W, feat_fc_b,
           dir_emb, weight, gcn_W1, gcn_b1, gcn_W2, gcn_b2, gcn_W3, gcn_b3,
           cls_W, cls_b, reg_W, reg_b):
    src = edge_index[0].astype(jnp.int32)
    dst = edge_index[1].astype(jnp.int32)
    npad = _EP - _E
    pad_src = (jnp.arange(npad, dtype=jnp.int32) * 67) % _N
    pad_dst = _N + (jnp.arange(npad, dtype=jnp.int32) % (_SPR - _N))
    src_p = jnp.concatenate([src, pad_src])
    dst3 = jnp.concatenate([dst, pad_dst])
    src4 = jnp.concatenate(
        [src_p + (qq * _N) for qq in range(_NQ)]).reshape(_NQ * _EP // _CH, _CH)
    index = index.astype(jnp.int32)
    idx4 = jnp.concatenate([index + (qq * _N) for qq in range(_NQ)])
    id2 = id.astype(jnp.int32)[:, None]

    deg2f, xinfo, xtext, xtitle = _sc_prep(
        dst3, info_index.astype(jnp.int32), text_index.astype(jnp.int32),
        index, x)
    deg2 = deg2f.reshape(_NC, _SPR, _FQ)

    w1q = gcn_W1.T.reshape(_NQ, _FQ, _D).transpose(0, 2, 1)
    hw1, dinv16 = _tc_first(x, deg2, feat_fc_W, feat_fc_b[None, :], w1q)

    base = _tc_heads(xinfo, xtext, xtitle, id2, dir_emb,
                     text_fc_W, text_fc_b[None, :],
                     image_fc_W, image_fc_b[None, :],
                     feat_fc_W, feat_fc_b[None, :], weight)

    eye8 = jnp.eye(8, dtype=jnp.float32)
    wb2 = gcn_W2.reshape(_NQ, _FQ, _NQ, _FQ)
    wb3 = gcn_W3.reshape(_NQ, _FQ, _NQ, _FQ)
    kw2 = jnp.einsum("ab,qiro->qraibo", eye8, wb2).reshape(
        _NQ, _NQ, 8 * _FQ, 8 * _FQ)
    kw3 = jnp.einsum("ab,qiro->qraibo", eye8, wb3).reshape(
        _NQ, _NQ, 8 * _FQ, 8 * _FQ)
    b1t = jnp.tile(gcn_b1.reshape(_NQ, 1, _FQ), (1, 1, 8))
    b2t = jnp.tile(gcn_b2.reshape(_NQ, 1, _FQ), (1, 1, 8))
    dinvf = dinv16.reshape(_N // 8, 8 * _FQ)
    hw1f = hw1.reshape(_NQ, _N // 8, 8 * _FQ)
    seg1 = _sc_segsum(src4, dst3, hw1f.reshape(_NQ * _N, _FQ))
    hw2 = _tc_layer(seg1.reshape(_NQ, _N // 8, 8 * _FQ), hw1f, dinvf, b1t, kw2)
    seg2 = _sc_segsum(src4, dst3, hw2.reshape(_NQ * _N, _FQ))
    hw3 = _tc_layer(seg2.reshape(_NQ, _N // 8, 8 * _FQ), hw2, dinvf, b2t, kw3)
    seg3 = _sc_segsum(src4, dst3, hw3.reshape(_NQ * _N, _FQ))

    segg, hwg, dinvg = _sc_final_gather(
        idx4, index, seg3, hw3.reshape(_NQ * _N, _FQ), dinv16)
    oc, orr = _tc_final(segg.reshape(_NQ, _B, _FQ),
                        hwg.reshape(_NQ, _B, _FQ),
                        dinvg, base, gcn_b3.reshape(_NQ, 1, _FQ),
                        cls_W, cls_W.reshape(_NQ, _FQ, _OUT), cls_b[None, :],
                        reg_W, reg_W.reshape(_NQ, _FQ, 1), reg_b[None, :])
    return (oc, orr)
